# baseline (speedup 1.0000x reference)
import jax
import jax.numpy as jnp
from jax.experimental import pallas as pl


def _copy_block(e_ref, o_ref):
    o_ref[...] = e_ref[...]


def kernel(x, encoding):
    seq_len = x.shape[1]
    n_embd = encoding.shape[1]
    block_rows = 512
    grid = (seq_len // block_rows,)
    return pl.pallas_call(
        _copy_block,
        grid=grid,
        in_specs=[pl.BlockSpec((block_rows, n_embd), lambda i: (i, 0))],
        out_specs=pl.BlockSpec((block_rows, n_embd), lambda i: (i, 0)),
        out_shape=jax.ShapeDtypeStruct((seq_len, n_embd), encoding.dtype),
    )(encoding)
